# baseline (speedup 1.0000x reference)
import jax
import jax.numpy as jnp
from jax.experimental import pallas as pl
from jax.experimental.pallas import tpu as pltpu

N_EMBED = 768
NUM_EXPERTS = 64
NEG_INF = float("-inf")


def _router_body(x_ref, w_ref, b_ref, out_ref, idx_ref):
    x = x_ref[0]
    bt = x.shape[0]
    g = bt // 64
    logits = jax.lax.dot_general(
        x, w_ref[...], (((1,), (1,)), ((), ())),
        preferred_element_type=jnp.float32) + b_ref[...]
    lg = logits.reshape(g, 64, NUM_EXPERTS)
    eiota = jax.lax.broadcasted_iota(jnp.int32, lg.shape, 2)
    i1 = jnp.argmax(lg, axis=-1)
    is1 = eiota == i1[..., None]
    m1 = jnp.max(lg, axis=-1, keepdims=True)
    masked = jnp.where(is1, NEG_INF, lg)
    i2 = jnp.argmax(masked, axis=-1)
    is2 = eiota == i2[..., None]
    m2 = jnp.max(masked, axis=-1, keepdims=True)
    e = jnp.exp(m2 - m1)
    denom = 1.0 + e
    p1 = 1.0 / denom
    p2 = e / denom
    out = jnp.where(is1, p1, 0.0) + jnp.where(is2, p2, 0.0)
    out_ref[0] = out.reshape(bt, NUM_EXPERTS)
    m_iota = jax.lax.broadcasted_iota(jnp.int32, (64, 128), 0)
    c_iota = jax.lax.broadcasted_iota(jnp.int32, (64, 128), 1)
    sel1 = (c_iota == 2 * m_iota).astype(jnp.float32)
    sel2 = (c_iota == 2 * m_iota + 1).astype(jnp.float32)
    packed = (
        jax.lax.dot_general(i1.astype(jnp.float32), sel1,
                            (((1,), (0,)), ((), ())),
                            preferred_element_type=jnp.float32)
        + jax.lax.dot_general(i2.astype(jnp.float32), sel2,
                              (((1,), (0,)), ((), ())),
                              preferred_element_type=jnp.float32)
    )
    idx_ref[0] = packed.astype(jnp.int32)


def kernel(mh_output, W, b):
    B, S, D = mh_output.shape
    b2 = b.reshape(1, NUM_EXPERTS)

    BT = 4096
    grid = (B, S // BT)
    out, idx_packed = pl.pallas_call(
        _router_body,
        grid=grid,
        in_specs=[
            pl.BlockSpec((1, BT, D), lambda i, j: (i, j, 0)),
            pl.BlockSpec((NUM_EXPERTS, D), lambda i, j: (0, 0)),
            pl.BlockSpec((1, NUM_EXPERTS), lambda i, j: (0, 0)),
        ],
        out_specs=[
            pl.BlockSpec((1, BT, NUM_EXPERTS), lambda i, j: (i, j, 0)),
            pl.BlockSpec((1, BT // 64, 128), lambda i, j: (i, j, 0)),
        ],
        out_shape=[
            jax.ShapeDtypeStruct((B, S, NUM_EXPERTS), jnp.float32),
            jax.ShapeDtypeStruct((B, S // 64, 128), jnp.int32),
        ],
        compiler_params=pltpu.CompilerParams(
            dimension_semantics=("parallel", "parallel"),
        ),
    )(mh_output, W, b2)
    idx = idx_packed.reshape(B, S, 2)
    return out, idx
